# baseline (speedup 1.0000x reference)
import functools

import jax
import jax.numpy as jnp
from jax import lax
from jax.experimental import pallas as pl
from jax.experimental.pallas import tpu as pltpu
from jax.experimental.pallas import tpu_sc as plsc

NC = 2
NS = 16
NW = NC * NS
L = 16

C = 512
G = C // L
NSET = 2


def _softplus(x):
    z = jnp.exp(-jnp.abs(x))
    t = z / (z + 2.0)
    t2 = t * t
    p = t * (2.0 + t2 * (2.0 / 3.0 + t2 * (2.0 / 5.0 + t2 * (2.0 / 7.0))))
    return jnp.maximum(x, 0.0) + p


@functools.lru_cache(maxsize=None)
def _build_sc_loss(NU, NI, E):
    n_per_w = -(-(-(-E // C)) // NW)
    n_per_w = -(-n_per_w // NSET) * NSET
    mesh = plsc.VectorSubcoreMesh(core_axis_name="core", subcore_axis_name="sub")
    scratch = (
        [pltpu.VMEM((C,), jnp.int32) for _ in range(8 * NSET)]
        + [pltpu.VMEM((2 * C,), jnp.int32) for _ in range(2 * NSET)]
        + [pltpu.VMEM((2 * C,), jnp.float32) for _ in range(2 * NSET)]
        + [pltpu.VMEM((L,), jnp.float32)]
        + [pltpu.SemaphoreType.DMA for _ in range(2 * NSET)]
    )

    @functools.partial(
        pl.kernel,
        mesh=mesh,
        out_type=jax.ShapeDtypeStruct((NW, L), jnp.float32),
        scratch_types=scratch,
    )
    def k(rate_hbm, link_hbm, pu, pi, pu1, pu2, nu, ni, nu1, nu2, out_hbm, *s):
        st = [s[8 * t:8 * t + 8] for t in range(NSET)]
        b0 = 8 * NSET
        fl = [s[b0 + 2 * t:b0 + 2 * t + 2] for t in range(NSET)]
        b1 = b0 + 2 * NSET
        dv = [s[b1 + 2 * t:b1 + 2 * t + 2] for t in range(NSET)]
        accv = s[b1 + 2 * NSET]
        sem_st = s[b1 + 2 * NSET + 1:b1 + 2 * NSET + 1 + NSET]
        sem_g = s[b1 + 2 * NSET + 1 + NSET:]
        w = lax.axis_index("sub") * NC + lax.axis_index("core")
        lane = lax.iota(jnp.int32, L)
        streams = ((pu, pi, NI, 0, 0), (nu, ni, NI, 0, C),
                   (pu1, pu2, NU, 1, 0), (nu1, nu2, NU, 1, C))
        tabs = (rate_hbm, link_hbm)

        def pair(q, acc):
            cids = [(q * NSET + t) * NW + w for t in range(NSET)]
            offs = [pl.multiple_of(jnp.minimum(cid * C, E - C), 8) for cid in cids]
            hs = []
            for t in range(NSET):
                hset = []
                for si, (ua, ia, _, _, _) in enumerate(streams):
                    hset.append(pltpu.async_copy(ua.at[pl.ds(offs[t], C)], st[t][2 * si], sem_st[t]))
                    hset.append(pltpu.async_copy(ia.at[pl.ds(offs[t], C)], st[t][2 * si + 1], sem_st[t]))
                hs.append(hset)
            gh = []
            for t in range(NSET):
                for h in hs[t]:
                    h.wait()

                def fbody(g, carry, t=t):
                    for si, (_, _, mult, fb, half) in enumerate(streams):
                        uv = st[t][2 * si][pl.ds(g * L, L)]
                        iv = st[t][2 * si + 1][pl.ds(g * L, L)]
                        fl[t][fb][pl.ds(half + g * L, L)] = (
                            (uv >> 3) * (mult * 8)
                            + (iv >> 7) * 1024
                            + (uv & 7) * 128
                            + (iv & 127)
                        )
                    return carry

                lax.fori_loop(0, G, fbody, 0)
                gh.append([
                    pltpu.async_copy(tabs[b].at[fl[t][b]], dv[t][b], sem_g[t])
                    for b in range(2)
                ])
            for t in range(NSET):
                for h in gh[t]:
                    h.wait()

                def gbody(g, a, t=t):
                    pr = dv[t][0][pl.ds(g * L, L)]
                    nr = dv[t][0][pl.ds(C + g * L, L)]
                    plk = dv[t][1][pl.ds(g * L, L)]
                    nlk = dv[t][1][pl.ds(C + g * L, L)]
                    gidx = offs[t] + g * L + lane
                    m = (gidx >= cids[t] * C) & (gidx < E)
                    term = _softplus(nr - pr) + _softplus(nlk - plk)
                    return a + jnp.where(m, term, 0.0)

                acc = lax.fori_loop(0, G, gbody, acc)
            return acc

        acc = lax.fori_loop(0, n_per_w // NSET, pair, jnp.zeros((L,), jnp.float32))
        accv[...] = acc
        pltpu.sync_copy(accv, out_hbm.at[w])

    return k


def kernel(rate_pred, link_pred, pos_u, pos_i, pos_u1, pos_u2, neg_u, neg_i, neg_u1, neg_u2):
    NU, NI = rate_pred.shape
    E = pos_u.shape[0]
    rate_flat = (rate_pred.reshape(NU // 8, 8, NI // 128, 128)
                 .transpose(0, 2, 1, 3).reshape(-1))
    link_flat = (link_pred.reshape(NU // 8, 8, NU // 128, 128)
                 .transpose(0, 2, 1, 3).reshape(-1))
    k = _build_sc_loss(NU, NI, E)
    parts = k(rate_flat, link_flat, pos_u, pos_i, pos_u1, pos_u2,
              neg_u, neg_i, neg_u1, neg_u2)
    return jnp.sum(parts)


# speedup vs baseline: 1.1029x; 1.1029x over previous
import functools

import jax
import jax.numpy as jnp
from jax import lax
from jax.experimental import pallas as pl
from jax.experimental.pallas import tpu as pltpu
from jax.experimental.pallas import tpu_sc as plsc

NC = 2
NS = 16
NW = NC * NS
L = 16

C = 1024
G = C // L
NSET = 4


def _softplus(x):
    z = jnp.exp(-jnp.abs(x))
    t = z / (z + 2.0)
    t2 = t * t
    p = t * (2.0 + t2 * (2.0 / 3.0 + t2 * (2.0 / 5.0 + t2 * (2.0 / 7.0))))
    return jnp.maximum(x, 0.0) + p


@functools.lru_cache(maxsize=None)
def _build_sc_loss(NU, NI, E):
    n_per_w = -(-(-(-E // C)) // NW)
    n_per_w = -(-n_per_w // NSET) * NSET
    mesh = plsc.VectorSubcoreMesh(core_axis_name="core", subcore_axis_name="sub")
    scratch = (
        [pltpu.VMEM((C,), jnp.int32) for _ in range(8 * NSET)]
        + [pltpu.VMEM((2 * C,), jnp.int32) for _ in range(2 * NSET)]
        + [pltpu.VMEM((2 * C,), jnp.float32) for _ in range(2 * NSET)]
        + [pltpu.VMEM((L,), jnp.float32)]
        + [pltpu.SemaphoreType.DMA for _ in range(2 * NSET)]
    )

    @functools.partial(
        pl.kernel,
        mesh=mesh,
        out_type=jax.ShapeDtypeStruct((NW, L), jnp.float32),
        scratch_types=scratch,
    )
    def k(rate_hbm, link_hbm, pu, pi, pu1, pu2, nu, ni, nu1, nu2, out_hbm, *s):
        st = [s[8 * t:8 * t + 8] for t in range(NSET)]
        b0 = 8 * NSET
        fl = [s[b0 + 2 * t:b0 + 2 * t + 2] for t in range(NSET)]
        b1 = b0 + 2 * NSET
        dv = [s[b1 + 2 * t:b1 + 2 * t + 2] for t in range(NSET)]
        accv = s[b1 + 2 * NSET]
        sem_st = s[b1 + 2 * NSET + 1:b1 + 2 * NSET + 1 + NSET]
        sem_g = s[b1 + 2 * NSET + 1 + NSET:]
        w = lax.axis_index("sub") * NC + lax.axis_index("core")
        lane = lax.iota(jnp.int32, L)
        streams = ((pu, pi, NI, 0, 0), (nu, ni, NI, 0, C),
                   (pu1, pu2, NU, 1, 0), (nu1, nu2, NU, 1, C))
        tabs = (rate_hbm, link_hbm)

        def pair(q, acc):
            cids = [(q * NSET + t) * NW + w for t in range(NSET)]
            offs = [pl.multiple_of(jnp.minimum(cid * C, E - C), 8) for cid in cids]
            hs = []
            for t in range(NSET):
                hset = []
                for si, (ua, ia, _, _, _) in enumerate(streams):
                    hset.append(pltpu.async_copy(ua.at[pl.ds(offs[t], C)], st[t][2 * si], sem_st[t]))
                    hset.append(pltpu.async_copy(ia.at[pl.ds(offs[t], C)], st[t][2 * si + 1], sem_st[t]))
                hs.append(hset)
            gh = []
            for t in range(NSET):
                for h in hs[t]:
                    h.wait()

                def fbody(g, carry, t=t):
                    for si, (_, _, mult, fb, half) in enumerate(streams):
                        uv = st[t][2 * si][pl.ds(g * L, L)]
                        iv = st[t][2 * si + 1][pl.ds(g * L, L)]
                        fl[t][fb][pl.ds(half + g * L, L)] = (
                            (uv >> 3) * (mult * 8)
                            + (iv >> 7) * 1024
                            + (uv & 7) * 128
                            + (iv & 127)
                        )
                    return carry

                lax.fori_loop(0, G, fbody, 0)
                gh.append([
                    pltpu.async_copy(tabs[b].at[fl[t][b]], dv[t][b], sem_g[t])
                    for b in range(2)
                ])
            for t in range(NSET):
                for h in gh[t]:
                    h.wait()

                def gbody(g, a, t=t):
                    pr = dv[t][0][pl.ds(g * L, L)]
                    nr = dv[t][0][pl.ds(C + g * L, L)]
                    plk = dv[t][1][pl.ds(g * L, L)]
                    nlk = dv[t][1][pl.ds(C + g * L, L)]
                    gidx = offs[t] + g * L + lane
                    m = (gidx >= cids[t] * C) & (gidx < E)
                    term = _softplus(nr - pr) + _softplus(nlk - plk)
                    return a + jnp.where(m, term, 0.0)

                acc = lax.fori_loop(0, G, gbody, acc)
            return acc

        acc = lax.fori_loop(0, n_per_w // NSET, pair, jnp.zeros((L,), jnp.float32))
        accv[...] = acc
        pltpu.sync_copy(accv, out_hbm.at[w])

    return k


def kernel(rate_pred, link_pred, pos_u, pos_i, pos_u1, pos_u2, neg_u, neg_i, neg_u1, neg_u2):
    NU, NI = rate_pred.shape
    E = pos_u.shape[0]
    rate_flat = (rate_pred.reshape(NU // 8, 8, NI // 128, 128)
                 .transpose(0, 2, 1, 3).reshape(-1))
    link_flat = (link_pred.reshape(NU // 8, 8, NU // 128, 128)
                 .transpose(0, 2, 1, 3).reshape(-1))
    k = _build_sc_loss(NU, NI, E)
    parts = k(rate_flat, link_flat, pos_u, pos_i, pos_u1, pos_u2,
              neg_u, neg_i, neg_u1, neg_u2)
    return jnp.sum(parts)
